# baseline (speedup 1.0000x reference)
import jax
import jax.numpy as jnp
from jax import lax
from jax.experimental import pallas as pl
from jax.experimental.pallas import tpu as pltpu
from jax.experimental.pallas import tpu_sc as plsc

N = 10000
E = 320000
D = 128
OUT = 64

NC = 2
NS = 16
G = 128
ITERS = 80
EPAD = NC * NS * ITERS * G
SG = 128
SITERS = 80
SPER_SUB = SG * SITERS
NACC = N + 8
ROWS_SUB = 640
ROWS_LAST = N - ROWS_SUB * (NS - 1)

BN = 1000
f32 = jnp.float32


def _zero_fill(zbuf):
    z16 = jnp.zeros((16,), f32)
    for r in range(16):
        for c in range(D // 16):
            zbuf[r, pl.ds(c * 16, 16)] = z16


def _acc_chunks(sid):
    nz = jnp.where(sid < NS - 1, ROWS_SUB // 16, ROWS_LAST // 16)
    nd = jnp.where(sid < NS - 1, ROWS_SUB // 80, ROWS_LAST // 80)
    return sid * ROWS_SUB, nz, nd


def _segsum_body(table, src, dst, s_out,
                 srcg0, srcg1, dstg0, dstg1, rows0, rows1, zbuf, acc,
                 gs0, gs1):
    cid = lax.axis_index("c")
    sid = lax.axis_index("s")
    wid = cid * NS + sid
    row0, nz, nd = _acc_chunks(sid)
    base0 = wid * SPER_SUB

    srcg = (srcg0, srcg1)
    dstg = (dstg0, dstg1)
    rows = (rows0, rows1)
    gsem = (gs0, gs1)

    def ld_src(i, p):
        pltpu.sync_copy(src.at[pl.ds(base0 + i * SG, SG)], srcg[p])

    def ld_dst(i, p):
        pltpu.sync_copy(dst.at[pl.ds(base0 + i * SG, SG)], dstg[p])

    def gat(b):
        pltpu.async_copy(table.at[srcg[b]], rows[b], gsem[b])

    def sca(b):
        pltpu.sync_copy(rows[b], acc.at[dstg[b]], add=True)

    def wait_g(b):
        pltpu.make_async_copy(table.at[srcg[b]], rows[b], gsem[b]).wait()

    _zero_fill(zbuf)
    lax.fori_loop(
        0, nz,
        lambda k, _: (pltpu.sync_copy(zbuf, acc.at[pl.ds(row0 + k * 16, 16)]),
                      0)[1], 0)
    plsc.subcore_barrier()

    ld_src(0, 0)
    gat(0)

    def pair(o, _):
        i0 = 2 * o
        ld_dst(i0, 0)
        ld_src(i0 + 1, 1)
        wait_g(0)
        gat(1)
        sca(0)
        ld_dst(i0 + 1, 1)

        @pl.when(i0 + 2 < SITERS)
        def _():
            ld_src(i0 + 2, 0)
        wait_g(1)

        @pl.when(i0 + 2 < SITERS)
        def _():
            gat(0)
        sca(1)
        return 0
    lax.fori_loop(0, SITERS // 2, pair, 0)

    if SITERS % 2:
        ld_dst(SITERS - 1, 0)
        wait_g(0)
        sca(0)

    plsc.subcore_barrier()

    def drain(k, _):
        b = row0 + k * 80
        pltpu.sync_copy(acc.at[pl.ds(b, 80)], s_out.at[cid, pl.ds(b, 80)])
        return 0
    lax.fori_loop(0, nd, drain, 0)


def _deg_body(dst2, deg_out, dstl, ones_v, zbuf, acc, ss0, ss1):
    cid = lax.axis_index("c")
    sid = lax.axis_index("s")
    wid = cid * NS + sid
    row0, nz, nd = _acc_chunks(sid)

    pltpu.sync_copy(dst2.at[wid], dstl)

    _zero_fill(zbuf)
    one16 = jnp.ones((16,), f32)
    for r in range(G):
        for c in range(D // 16):
            ones_v[r, pl.ds(c * 16, 16)] = one16
    lax.fori_loop(
        0, nz,
        lambda k, _: (pltpu.sync_copy(zbuf, acc.at[pl.ds(row0 + k * 16, 16)]),
                      0)[1], 0)
    plsc.subcore_barrier()

    ssem = (ss0, ss1)

    def sca(i, b):
        return pltpu.async_copy(ones_v, acc.at[dstl.at[i]], ssem[b],
                                add=True)

    def wait_s(b):
        pltpu.make_async_copy(ones_v, acc.at[dstl.at[0]], ssem[b]).wait()

    sca(0, 0)
    sca(1, 1)

    def pair(o, _):
        wait_s(0)
        sca(2 * o + 2, 0)
        wait_s(1)
        sca(2 * o + 3, 1)
        return 0
    lax.fori_loop(0, ITERS // 2 - 1, pair, 0)
    wait_s(0)
    wait_s(1)

    plsc.subcore_barrier()

    def drain(k, _):
        b = row0 + k * 80
        pltpu.sync_copy(acc.at[pl.ds(b, 80)], deg_out.at[cid, pl.ds(b, 80)])
        return 0
    lax.fori_loop(0, nd, drain, 0)


_MESH = plsc.VectorSubcoreMesh(core_axis_name="c", subcore_axis_name="s",
                               num_cores=NC, num_subcores=NS)

_segsum = pl.kernel(
    _segsum_body,
    out_type=jax.ShapeDtypeStruct((NC, N, D), f32),
    mesh=_MESH,
    scratch_types=(pltpu.VMEM((SG,), jnp.int32),
                   pltpu.VMEM((SG,), jnp.int32),
                   pltpu.VMEM((SG,), jnp.int32),
                   pltpu.VMEM((SG,), jnp.int32),
                   pltpu.VMEM((SG, D), f32),
                   pltpu.VMEM((SG, D), f32),
                   pltpu.VMEM((16, D), f32),
                   pltpu.VMEM_SHARED((NACC, D), f32))
    + (pltpu.SemaphoreType.DMA,) * 2)

_deg_count = pl.kernel(
    _deg_body,
    out_type=jax.ShapeDtypeStruct((NC, N, D), f32),
    mesh=_MESH,
    scratch_types=(pltpu.VMEM((ITERS, G), jnp.int32),
                   pltpu.VMEM((G, D), f32),
                   pltpu.VMEM((16, D), f32),
                   pltpu.VMEM_SHARED((NACC, D), f32),
                   pltpu.SemaphoreType.DMA,
                   pltpu.SemaphoreType.DMA))


def _stage_a_body(tau_ref, x_ref, sxp_ref, degp_ref, h_ref, dinv_ref):
    deg = jnp.maximum(degp_ref[0, :, 0] + degp_ref[1, :, 0], 1.0)
    dinv = (1.0 / deg)[:, None]
    s = sxp_ref[0] + sxp_ref[1]
    xb = x_ref[...]
    h_ref[...] = xb + tau_ref[0, 0] * (s * dinv - xb)
    dinv_ref[...] = jnp.broadcast_to(dinv, (dinv.shape[0], 8))


def _stage_bc_body(h_ref, sp_ref, dinv_ref, w_ref, g_ref):
    dinv = dinv_ref[:, 0][:, None]
    msg = (sp_ref[0] + sp_ref[1]) * dinv - h_ref[...]
    g_ref[...] = jnp.tanh(jnp.dot(msg, w_ref[...],
                                  precision=lax.Precision.HIGHEST))


def _stage_d_body(g1_ref, g2_ref, s1p_ref, s2p_ref, dinv_ref,
                  wconv_ref, wmlp_ref, out_ref):
    dinv = dinv_ref[:, 0][:, None]
    cat = jnp.concatenate(
        [g1_ref[...], g2_ref[...],
         (s1p_ref[0] + s1p_ref[1]) * dinv,
         (s2p_ref[0] + s2p_ref[1]) * dinv], axis=1)
    pre = jax.nn.relu(jnp.dot(cat, wconv_ref[...],
                              precision=lax.Precision.HIGHEST))
    out_ref[...] = jnp.dot(pre, wmlp_ref[...],
                           precision=lax.Precision.HIGHEST)


def _blk():
    return pl.BlockSpec((BN, D), lambda i: (i, 0))


def _pblk():
    return pl.BlockSpec((NC, BN, D), lambda i: (0, i, 0))


def _dinv_blk():
    return pl.BlockSpec((BN, 8), lambda i: (i, 0))


def _full(shape):
    return pl.BlockSpec(shape, lambda i: (0, 0))


_GRID = (N // BN,)


def _stage_a(x, sxp, degp, tau2):
    return pl.pallas_call(
        _stage_a_body,
        grid=_GRID,
        in_specs=[pl.BlockSpec(memory_space=pltpu.SMEM),
                  _blk(), _pblk(), _pblk()],
        out_specs=[_blk(), _dinv_blk()],
        out_shape=[jax.ShapeDtypeStruct((N, D), f32),
                   jax.ShapeDtypeStruct((N, 8), f32)],
    )(tau2, x, sxp, degp)


def _stage_bc(h, sp, dinv, w):
    return pl.pallas_call(
        _stage_bc_body,
        grid=_GRID,
        in_specs=[_blk(), _pblk(), _dinv_blk(), _full((D, D))],
        out_specs=_blk(),
        out_shape=jax.ShapeDtypeStruct((N, D), f32),
    )(h, sp, dinv, w)


def _stage_d(g1, g2, s1p, s2p, dinv, wconv, wmlp):
    return pl.pallas_call(
        _stage_d_body,
        grid=_GRID,
        in_specs=[_blk(), _blk(), _pblk(), _pblk(), _dinv_blk(),
                  _full((4 * D, D)), _full((D, OUT))],
        out_specs=pl.BlockSpec((BN, OUT), lambda i: (i, 0)),
        out_shape=jax.ShapeDtypeStruct((N, OUT), f32),
    )(g1, g2, s1p, s2p, dinv, wconv, wmlp)


def kernel(x, edge_index, tau, Wg0, Wg1, Wconv, Wmlp):
    src = edge_index[0].astype(jnp.int32)
    dst = edge_index[1].astype(jnp.int32)
    src = jnp.concatenate([src, jnp.zeros((EPAD - E,), jnp.int32)])
    dst_p = jnp.concatenate([dst, jnp.full((EPAD - E,), N, jnp.int32)])
    dst2 = dst_p.reshape(NC * NS, ITERS, G)
    dst = dst_p
    tau2 = jnp.reshape(tau, (1, 1)).astype(f32)

    degp = _deg_count(dst2)
    sxp = _segsum(x, src, dst)
    h, dinv = _stage_a(x, sxp, degp, tau2)
    shp = _segsum(h, src, dst)
    g1 = _stage_bc(h, shp, dinv, Wg0)
    sg1p = _segsum(g1, src, dst)
    g2 = _stage_bc(g1, sg1p, dinv, Wg1)
    sg2p = _segsum(g2, src, dst)
    return _stage_d(g1, g2, sg1p, sg2p, dinv, Wconv, Wmlp)


# speedup vs baseline: 1.0004x; 1.0004x over previous
import jax
import jax.numpy as jnp
from jax import lax
from jax.experimental import pallas as pl
from jax.experimental.pallas import tpu as pltpu
from jax.experimental.pallas import tpu_sc as plsc

N = 10000
E = 320000
D = 128
OUT = 64

NC = 2
NS = 16
G = 128
ITERS = 80
EPAD = NC * NS * ITERS * G
SG = 128
SITERS = 80
SPER_SUB = SG * SITERS
NACC = N + 8
ROWS_SUB = 640
ROWS_LAST = N - ROWS_SUB * (NS - 1)

BN = 1000
f32 = jnp.float32


def _zero_fill(zbuf):
    z16 = jnp.zeros((16,), f32)
    for r in range(16):
        for c in range(D // 16):
            zbuf[r, pl.ds(c * 16, 16)] = z16


def _acc_chunks(sid):
    nz = jnp.where(sid < NS - 1, ROWS_SUB // 16, ROWS_LAST // 16)
    nd = jnp.where(sid < NS - 1, ROWS_SUB // 80, ROWS_LAST // 80)
    return sid * ROWS_SUB, nz, nd


def _segsum_body(table, src, dst, s_out,
                 srcg0, srcg1, dstg0, dstg1, rows0, rows1, zbuf, acc,
                 gs0, gs1):
    cid = lax.axis_index("c")
    sid = lax.axis_index("s")
    wid = cid * NS + sid
    row0, nz, nd = _acc_chunks(sid)
    base0 = wid * SPER_SUB

    srcg = (srcg0, srcg1)
    dstg = (dstg0, dstg1)
    rows = (rows0, rows1)
    gsem = (gs0, gs1)

    def ld_src(i, p):
        pltpu.sync_copy(src.at[pl.ds(base0 + i * SG, SG)], srcg[p])

    def ld_dst(i, p):
        pltpu.sync_copy(dst.at[pl.ds(base0 + i * SG, SG)], dstg[p])

    def gat(b):
        pltpu.async_copy(table.at[srcg[b]], rows[b], gsem[b])

    def sca(b):
        pltpu.sync_copy(rows[b], acc.at[dstg[b]], add=True)

    def wait_g(b):
        pltpu.make_async_copy(table.at[srcg[b]], rows[b], gsem[b]).wait()

    _zero_fill(zbuf)
    lax.fori_loop(
        0, nz,
        lambda k, _: (pltpu.sync_copy(zbuf, acc.at[pl.ds(row0 + k * 16, 16)]),
                      0)[1], 0)
    plsc.subcore_barrier()

    ld_src(0, 0)
    gat(0)

    def pair(o, _):
        i0 = 2 * o
        ld_dst(i0, 0)
        ld_src(i0 + 1, 1)
        wait_g(0)
        gat(1)
        sca(0)
        ld_dst(i0 + 1, 1)

        @pl.when(i0 + 2 < SITERS)
        def _():
            ld_src(i0 + 2, 0)
        wait_g(1)

        @pl.when(i0 + 2 < SITERS)
        def _():
            gat(0)
        sca(1)
        return 0
    lax.fori_loop(0, SITERS // 2, pair, 0)

    if SITERS % 2:
        ld_dst(SITERS - 1, 0)
        wait_g(0)
        sca(0)

    plsc.subcore_barrier()

    def drain(k, _):
        b = row0 + k * 80
        pltpu.sync_copy(acc.at[pl.ds(b, 80)], s_out.at[cid, pl.ds(b, 80)])
        return 0
    lax.fori_loop(0, nd, drain, 0)


def _deg_body(dst2, deg_out, dstl, ones_v, zbuf, acc, ss0, ss1):
    cid = lax.axis_index("c")
    sid = lax.axis_index("s")
    wid = cid * NS + sid
    row0, nz, nd = _acc_chunks(sid)

    pltpu.sync_copy(dst2.at[wid], dstl)

    _zero_fill(zbuf)
    one16 = jnp.ones((16,), f32)
    for r in range(G):
        for c in range(D // 16):
            ones_v[r, pl.ds(c * 16, 16)] = one16
    lax.fori_loop(
        0, nz,
        lambda k, _: (pltpu.sync_copy(zbuf, acc.at[pl.ds(row0 + k * 16, 16)]),
                      0)[1], 0)
    plsc.subcore_barrier()

    ssem = (ss0, ss1)

    def sca(i, b):
        return pltpu.async_copy(ones_v, acc.at[dstl.at[i]], ssem[b],
                                add=True)

    def wait_s(b):
        pltpu.make_async_copy(ones_v, acc.at[dstl.at[0]], ssem[b]).wait()

    sca(0, 0)
    sca(1, 1)

    def pair(o, _):
        wait_s(0)
        sca(2 * o + 2, 0)
        wait_s(1)
        sca(2 * o + 3, 1)
        return 0
    lax.fori_loop(0, ITERS // 2 - 1, pair, 0)
    wait_s(0)
    wait_s(1)

    plsc.subcore_barrier()

    def drain(k, _):
        b = row0 + k * 80
        pltpu.sync_copy(acc.at[pl.ds(b, 80)], deg_out.at[cid, pl.ds(b, 80)])
        return 0
    lax.fori_loop(0, nd, drain, 0)


_MESH = plsc.VectorSubcoreMesh(core_axis_name="c", subcore_axis_name="s",
                               num_cores=NC, num_subcores=NS)

_segsum = pl.kernel(
    _segsum_body,
    out_type=jax.ShapeDtypeStruct((NC, N, D), f32),
    mesh=_MESH,
    scratch_types=(pltpu.VMEM((SG,), jnp.int32),
                   pltpu.VMEM((SG,), jnp.int32),
                   pltpu.VMEM((SG,), jnp.int32),
                   pltpu.VMEM((SG,), jnp.int32),
                   pltpu.VMEM((SG, D), f32),
                   pltpu.VMEM((SG, D), f32),
                   pltpu.VMEM((16, D), f32),
                   pltpu.VMEM_SHARED((NACC, D), f32))
    + (pltpu.SemaphoreType.DMA,) * 2)

_deg_count = pl.kernel(
    _deg_body,
    out_type=jax.ShapeDtypeStruct((NC, N, D), f32),
    mesh=_MESH,
    scratch_types=(pltpu.VMEM((ITERS, G), jnp.int32),
                   pltpu.VMEM((G, D), f32),
                   pltpu.VMEM((16, D), f32),
                   pltpu.VMEM_SHARED((NACC, D), f32),
                   pltpu.SemaphoreType.DMA,
                   pltpu.SemaphoreType.DMA))


def _stage_a_body(tau_ref, x_ref, sxp_ref, degp_ref, h_ref, dinv_ref):
    deg = jnp.maximum(degp_ref[0, :, 0] + degp_ref[1, :, 0], 1.0)
    dinv = (1.0 / deg)[:, None]
    s = sxp_ref[0] + sxp_ref[1]
    xb = x_ref[...]
    h_ref[...] = xb + tau_ref[0, 0] * (s * dinv - xb)
    dinv_ref[...] = jnp.broadcast_to(dinv, (dinv.shape[0], 8))


def _stage_bc_body(h_ref, sp_ref, dinv_ref, w_ref, g_ref):
    dinv = dinv_ref[:, 0][:, None]
    msg = (sp_ref[0] + sp_ref[1]) * dinv - h_ref[...]
    g_ref[...] = jnp.tanh(jnp.dot(msg, w_ref[...],
                                  precision=lax.Precision.HIGHEST))


def _stage_d_body(g1_ref, g2_ref, s1p_ref, s2p_ref, dinv_ref,
                  wconv_ref, wmlp_ref, out_ref):
    dinv = dinv_ref[:, 0][:, None]
    cat = jnp.concatenate(
        [g1_ref[...], g2_ref[...],
         (s1p_ref[0] + s1p_ref[1]) * dinv,
         (s2p_ref[0] + s2p_ref[1]) * dinv], axis=1)
    pre = jax.nn.relu(jnp.dot(cat, wconv_ref[...],
                              precision=lax.Precision.HIGHEST))
    out_ref[...] = jnp.dot(pre, wmlp_ref[...],
                           precision=lax.Precision.HIGHEST)


def _blk():
    return pl.BlockSpec((BN, D), lambda i: (i, 0))


def _pblk():
    return pl.BlockSpec((NC, BN, D), lambda i: (0, i, 0))


def _dinv_blk():
    return pl.BlockSpec((BN, 8), lambda i: (i, 0))


def _full(shape):
    return pl.BlockSpec(shape, lambda i: (0, 0))


_GRID = (N // BN,)


def _stage_a(x, sxp, degp, tau2):
    return pl.pallas_call(
        _stage_a_body,
        grid=_GRID,
        in_specs=[pl.BlockSpec(memory_space=pltpu.SMEM),
                  _blk(), _pblk(), _pblk()],
        out_specs=[_blk(), _dinv_blk()],
        out_shape=[jax.ShapeDtypeStruct((N, D), f32),
                   jax.ShapeDtypeStruct((N, 8), f32)],
    )(tau2, x, sxp, degp)


def _stage_bc(h, sp, dinv, w):
    return pl.pallas_call(
        _stage_bc_body,
        grid=_GRID,
        in_specs=[_blk(), _pblk(), _dinv_blk(), _full((D, D))],
        out_specs=_blk(),
        out_shape=jax.ShapeDtypeStruct((N, D), f32),
    )(h, sp, dinv, w)


def _stage_d(g1, g2, s1p, s2p, dinv, wconv, wmlp):
    return pl.pallas_call(
        _stage_d_body,
        grid=_GRID,
        in_specs=[_blk(), _blk(), _pblk(), _pblk(), _dinv_blk(),
                  _full((4 * D, D)), _full((D, OUT))],
        out_specs=pl.BlockSpec((BN, OUT), lambda i: (i, 0)),
        out_shape=jax.ShapeDtypeStruct((N, OUT), f32),
    )(g1, g2, s1p, s2p, dinv, wconv, wmlp)


def kernel(x, edge_index, tau, Wg0, Wg1, Wconv, Wmlp):
    src = edge_index[0].astype(jnp.int32)
    dst = edge_index[1].astype(jnp.int32)
    src = jnp.concatenate([src, jnp.zeros((EPAD - E,), jnp.int32)])
    dst_p = jnp.concatenate(
        [dst, N + (jnp.arange(EPAD - E, dtype=jnp.int32) % 8)])
    dst2 = dst_p.reshape(NC * NS, ITERS, G)
    dst = dst_p
    tau2 = jnp.reshape(tau, (1, 1)).astype(f32)

    degp = _deg_count(dst2)
    sxp = _segsum(x, src, dst)
    h, dinv = _stage_a(x, sxp, degp, tau2)
    shp = _segsum(h, src, dst)
    g1 = _stage_bc(h, shp, dinv, Wg0)
    sg1p = _segsum(g1, src, dst)
    g2 = _stage_bc(g1, sg1p, dinv, Wg1)
    sg2p = _segsum(g2, src, dst)
    return _stage_d(g1, g2, sg1p, sg2p, dinv, Wconv, Wmlp)


# speedup vs baseline: 1.0394x; 1.0391x over previous
import jax
import jax.numpy as jnp
from jax import lax
from jax.experimental import pallas as pl
from jax.experimental.pallas import tpu as pltpu
from jax.experimental.pallas import tpu_sc as plsc

N = 10000
E = 320000
D = 128
OUT = 64

NC = 2
NS = 16
G = 128
ITERS = 80
EPAD = NC * NS * ITERS * G
SG = 80
SITERS = 128
SPER_SUB = SG * SITERS
NACC = N + 8
ROWS_SUB = 640
ROWS_LAST = N - ROWS_SUB * (NS - 1)

BN = 1000
f32 = jnp.float32


def _zero_fill(zbuf):
    z16 = jnp.zeros((16,), f32)
    for r in range(16):
        for c in range(D // 16):
            zbuf[r, pl.ds(c * 16, 16)] = z16


def _acc_chunks(sid):
    nz = jnp.where(sid < NS - 1, ROWS_SUB // 16, ROWS_LAST // 16)
    nd = jnp.where(sid < NS - 1, ROWS_SUB // 80, ROWS_LAST // 80)
    return sid * ROWS_SUB, nz, nd


NB = 4


def _segsum_body(table, src, dst, s_out,
                 srcg0, srcg1, srcg2, srcg3,
                 dstg0, dstg1, dstg2, dstg3,
                 rows0, rows1, rows2, rows3, acc,
                 gs0, gs1, gs2, gs3, ss0, ss1, ss2, ss3):
    cid = lax.axis_index("c")
    sid = lax.axis_index("s")
    wid = cid * NS + sid
    row0, nz, nd = _acc_chunks(sid)
    base0 = wid * SPER_SUB

    srcg = (srcg0, srcg1, srcg2, srcg3)
    dstg = (dstg0, dstg1, dstg2, dstg3)
    rows = (rows0, rows1, rows2, rows3)
    gsem = (gs0, gs1, gs2, gs3)
    ssem = (ss0, ss1, ss2, ss3)

    def ld_src(i, p):
        pltpu.sync_copy(src.at[pl.ds(base0 + i * SG, SG)], srcg[p])

    def ld_dst(i, p):
        pltpu.sync_copy(dst.at[pl.ds(base0 + i * SG, SG)], dstg[p])

    def gat(b):
        pltpu.async_copy(table.at[srcg[b]], rows[b], gsem[b])

    def sca(b):
        pltpu.async_copy(rows[b], acc.at[dstg[b]], ssem[b], add=True)

    def wait_g(b):
        pltpu.make_async_copy(table.at[srcg[b]], rows[b], gsem[b]).wait()

    def wait_s(b):
        pltpu.make_async_copy(rows[b], acc.at[dstg[b]], ssem[b]).wait()

    _zero_fill(rows0)
    lax.fori_loop(
        0, nz,
        lambda k, _: (pltpu.sync_copy(rows0.at[pl.ds(0, 16)],
                                      acc.at[pl.ds(row0 + k * 16, 16)]),
                      0)[1], 0)
    plsc.subcore_barrier()

    for i in range(NB - 1):
        ld_src(i, i)
        gat(i)

    def quad(o, _):
        for j in range(NB):
            i = NB * o + j
            nb = (j + NB - 1) % NB
            wait_g(j)
            ld_dst(i, j)
            sca(j)
            if j == 0:
                @pl.when(o >= 1)
                def _():
                    wait_s(nb)
                ld_src(i + NB - 1, nb)
                gat(nb)
            else:
                @pl.when(i + NB - 1 < SITERS)
                def _():
                    wait_s(nb)
                    ld_src(i + NB - 1, nb)
                    gat(nb)
        return 0
    lax.fori_loop(0, SITERS // NB, quad, 0)

    for b in range(NB):
        wait_s(b)

    plsc.subcore_barrier()

    def drain(k, _):
        b = row0 + k * 80
        pltpu.sync_copy(acc.at[pl.ds(b, 80)], s_out.at[cid, pl.ds(b, 80)])
        return 0
    lax.fori_loop(0, nd, drain, 0)


def _deg_body(dst2, deg_out, dstl, ones_v, zbuf, acc, ss0, ss1):
    cid = lax.axis_index("c")
    sid = lax.axis_index("s")
    wid = cid * NS + sid
    row0, nz, nd = _acc_chunks(sid)

    pltpu.sync_copy(dst2.at[wid], dstl)

    _zero_fill(zbuf)
    one16 = jnp.ones((16,), f32)
    for r in range(G):
        for c in range(D // 16):
            ones_v[r, pl.ds(c * 16, 16)] = one16
    lax.fori_loop(
        0, nz,
        lambda k, _: (pltpu.sync_copy(zbuf, acc.at[pl.ds(row0 + k * 16, 16)]),
                      0)[1], 0)
    plsc.subcore_barrier()

    ssem = (ss0, ss1)

    def sca(i, b):
        return pltpu.async_copy(ones_v, acc.at[dstl.at[i]], ssem[b],
                                add=True)

    def wait_s(b):
        pltpu.make_async_copy(ones_v, acc.at[dstl.at[0]], ssem[b]).wait()

    sca(0, 0)
    sca(1, 1)

    def pair(o, _):
        wait_s(0)
        sca(2 * o + 2, 0)
        wait_s(1)
        sca(2 * o + 3, 1)
        return 0
    lax.fori_loop(0, ITERS // 2 - 1, pair, 0)
    wait_s(0)
    wait_s(1)

    plsc.subcore_barrier()

    def drain(k, _):
        b = row0 + k * 80
        pltpu.sync_copy(acc.at[pl.ds(b, 80)], deg_out.at[cid, pl.ds(b, 80)])
        return 0
    lax.fori_loop(0, nd, drain, 0)


_MESH = plsc.VectorSubcoreMesh(core_axis_name="c", subcore_axis_name="s",
                               num_cores=NC, num_subcores=NS)

_segsum = pl.kernel(
    _segsum_body,
    out_type=jax.ShapeDtypeStruct((NC, N, D), f32),
    mesh=_MESH,
    scratch_types=(pltpu.VMEM((SG,), jnp.int32),) * (2 * NB)
    + (pltpu.VMEM((SG, D), f32),) * NB
    + (pltpu.VMEM_SHARED((NACC, D), f32),)
    + (pltpu.SemaphoreType.DMA,) * (2 * NB))

_deg_count = pl.kernel(
    _deg_body,
    out_type=jax.ShapeDtypeStruct((NC, N, D), f32),
    mesh=_MESH,
    scratch_types=(pltpu.VMEM((ITERS, G), jnp.int32),
                   pltpu.VMEM((G, D), f32),
                   pltpu.VMEM((16, D), f32),
                   pltpu.VMEM_SHARED((NACC, D), f32),
                   pltpu.SemaphoreType.DMA,
                   pltpu.SemaphoreType.DMA))


def _stage_a_body(tau_ref, x_ref, sxp_ref, degp_ref, h_ref, dinv_ref):
    deg = jnp.maximum(degp_ref[0, :, 0] + degp_ref[1, :, 0], 1.0)
    dinv = (1.0 / deg)[:, None]
    s = sxp_ref[0] + sxp_ref[1]
    xb = x_ref[...]
    h_ref[...] = xb + tau_ref[0, 0] * (s * dinv - xb)
    dinv_ref[...] = jnp.broadcast_to(dinv, (dinv.shape[0], 8))


def _stage_bc_body(h_ref, sp_ref, dinv_ref, w_ref, g_ref):
    dinv = dinv_ref[:, 0][:, None]
    msg = (sp_ref[0] + sp_ref[1]) * dinv - h_ref[...]
    g_ref[...] = jnp.tanh(jnp.dot(msg, w_ref[...],
                                  precision=lax.Precision.HIGHEST))


def _stage_d_body(g1_ref, g2_ref, s1p_ref, s2p_ref, dinv_ref,
                  wconv_ref, wmlp_ref, out_ref):
    dinv = dinv_ref[:, 0][:, None]
    cat = jnp.concatenate(
        [g1_ref[...], g2_ref[...],
         (s1p_ref[0] + s1p_ref[1]) * dinv,
         (s2p_ref[0] + s2p_ref[1]) * dinv], axis=1)
    pre = jax.nn.relu(jnp.dot(cat, wconv_ref[...],
                              precision=lax.Precision.HIGHEST))
    out_ref[...] = jnp.dot(pre, wmlp_ref[...],
                           precision=lax.Precision.HIGHEST)


def _blk():
    return pl.BlockSpec((BN, D), lambda i: (i, 0))


def _pblk():
    return pl.BlockSpec((NC, BN, D), lambda i: (0, i, 0))


def _dinv_blk():
    return pl.BlockSpec((BN, 8), lambda i: (i, 0))


def _full(shape):
    return pl.BlockSpec(shape, lambda i: (0, 0))


_GRID = (N // BN,)


def _stage_a(x, sxp, degp, tau2):
    return pl.pallas_call(
        _stage_a_body,
        grid=_GRID,
        in_specs=[pl.BlockSpec(memory_space=pltpu.SMEM),
                  _blk(), _pblk(), _pblk()],
        out_specs=[_blk(), _dinv_blk()],
        out_shape=[jax.ShapeDtypeStruct((N, D), f32),
                   jax.ShapeDtypeStruct((N, 8), f32)],
    )(tau2, x, sxp, degp)


def _stage_bc(h, sp, dinv, w):
    return pl.pallas_call(
        _stage_bc_body,
        grid=_GRID,
        in_specs=[_blk(), _pblk(), _dinv_blk(), _full((D, D))],
        out_specs=_blk(),
        out_shape=jax.ShapeDtypeStruct((N, D), f32),
    )(h, sp, dinv, w)


def _stage_d(g1, g2, s1p, s2p, dinv, wconv, wmlp):
    return pl.pallas_call(
        _stage_d_body,
        grid=_GRID,
        in_specs=[_blk(), _blk(), _pblk(), _pblk(), _dinv_blk(),
                  _full((4 * D, D)), _full((D, OUT))],
        out_specs=pl.BlockSpec((BN, OUT), lambda i: (i, 0)),
        out_shape=jax.ShapeDtypeStruct((N, OUT), f32),
    )(g1, g2, s1p, s2p, dinv, wconv, wmlp)


def kernel(x, edge_index, tau, Wg0, Wg1, Wconv, Wmlp):
    src = edge_index[0].astype(jnp.int32)
    dst = edge_index[1].astype(jnp.int32)
    src = jnp.concatenate([src, jnp.zeros((EPAD - E,), jnp.int32)])
    dst = jnp.concatenate(
        [dst, N + (jnp.arange(EPAD - E, dtype=jnp.int32) % 8)])
    dst2 = dst.reshape(NC * NS, ITERS, G)
    tau2 = jnp.reshape(tau, (1, 1)).astype(f32)

    degp = _deg_count(dst2)
    sxp = _segsum(x, src, dst)
    h, dinv = _stage_a(x, sxp, degp, tau2)
    shp = _segsum(h, src, dst)
    g1 = _stage_bc(h, shp, dinv, Wg0)
    sg1p = _segsum(g1, src, dst)
    g2 = _stage_bc(g1, sg1p, dinv, Wg1)
    sg2p = _segsum(g2, src, dst)
    return _stage_d(g1, g2, sg1p, sg2p, dinv, Wconv, Wmlp)


# speedup vs baseline: 1.2064x; 1.1606x over previous
import jax
import jax.numpy as jnp
from jax import lax
from jax.experimental import pallas as pl
from jax.experimental.pallas import tpu as pltpu
from jax.experimental.pallas import tpu_sc as plsc

N = 10000
E = 320000
D = 128
OUT = 64

NC = 2
NS = 16
G = 128
ITERS = 80
EPAD = NC * NS * ITERS * G
SG = 80
SITERS = 128
SPER_SUB = SG * SITERS
NACC = N + 256
ROWS_SUB = 640
ROWS_LAST = N - ROWS_SUB * (NS - 1)

BN = 1000
f32 = jnp.float32


def _zero_fill(zbuf):
    z16 = jnp.zeros((16,), f32)
    for r in range(16):
        for c in range(D // 16):
            zbuf[r, pl.ds(c * 16, 16)] = z16


def _acc_chunks(sid):
    nz = jnp.where(sid < NS - 1, ROWS_SUB // 16, ROWS_LAST // 16)
    nd = jnp.where(sid < NS - 1, ROWS_SUB // 80, ROWS_LAST // 80)
    return sid * ROWS_SUB, nz, nd


NB = 4


def _segsum_body(table, src, dst, s_out,
                 srcg0, srcg1, srcg2, srcg3,
                 dstg0, dstg1, dstg2, dstg3,
                 rows0, rows1, rows2, rows3, acc,
                 gs0, gs1, gs2, gs3, ss0, ss1, ss2, ss3):
    cid = lax.axis_index("c")
    sid = lax.axis_index("s")
    wid = cid * NS + sid
    row0, nz, nd = _acc_chunks(sid)
    base0 = wid * SPER_SUB

    srcg = (srcg0, srcg1, srcg2, srcg3)
    dstg = (dstg0, dstg1, dstg2, dstg3)
    rows = (rows0, rows1, rows2, rows3)
    gsem = (gs0, gs1, gs2, gs3)
    ssem = (ss0, ss1, ss2, ss3)

    def ld_src(i, p):
        pltpu.sync_copy(src.at[pl.ds(base0 + i * SG, SG)], srcg[p])

    def ld_dst(i, p):
        pltpu.sync_copy(dst.at[pl.ds(base0 + i * SG, SG)], dstg[p])

    def gat(b):
        pltpu.async_copy(table.at[srcg[b]], rows[b], gsem[b])

    def sca(b):
        pltpu.async_copy(rows[b], acc.at[dstg[b]], ssem[b], add=True)

    def wait_g(b):
        pltpu.make_async_copy(table.at[srcg[b]], rows[b], gsem[b]).wait()

    def wait_s(b):
        pltpu.make_async_copy(rows[b], acc.at[dstg[b]], ssem[b]).wait()

    _zero_fill(rows0)
    lax.fori_loop(
        0, nz,
        lambda k, _: (pltpu.sync_copy(rows0.at[pl.ds(0, 16)],
                                      acc.at[pl.ds(row0 + k * 16, 16)]),
                      0)[1], 0)
    plsc.subcore_barrier()

    for i in range(NB - 1):
        ld_src(i, i)
        gat(i)

    def quad(o, _):
        for j in range(NB):
            i = NB * o + j
            nb = (j + NB - 1) % NB
            wait_g(j)
            ld_dst(i, j)
            sca(j)
            if j == 0:
                @pl.when(o >= 1)
                def _():
                    wait_s(nb)
                ld_src(i + NB - 1, nb)
                gat(nb)
            else:
                @pl.when(i + NB - 1 < SITERS)
                def _():
                    wait_s(nb)
                    ld_src(i + NB - 1, nb)
                    gat(nb)
        return 0
    lax.fori_loop(0, SITERS // NB, quad, 0)

    for b in range(NB):
        wait_s(b)

    plsc.subcore_barrier()

    def drain(k, _):
        b = row0 + k * 80
        pltpu.sync_copy(acc.at[pl.ds(b, 80)], s_out.at[cid, pl.ds(b, 80)])
        return 0
    lax.fori_loop(0, nd, drain, 0)


def _deg_body(dst2, deg_out, dstl, ones_v, zbuf, acc, ss0, ss1):
    cid = lax.axis_index("c")
    sid = lax.axis_index("s")
    wid = cid * NS + sid
    row0, nz, nd = _acc_chunks(sid)

    pltpu.sync_copy(dst2.at[wid], dstl)

    _zero_fill(zbuf)
    one16 = jnp.ones((16,), f32)
    for r in range(G):
        for c in range(D // 16):
            ones_v[r, pl.ds(c * 16, 16)] = one16
    lax.fori_loop(
        0, nz,
        lambda k, _: (pltpu.sync_copy(zbuf, acc.at[pl.ds(row0 + k * 16, 16)]),
                      0)[1], 0)
    plsc.subcore_barrier()

    ssem = (ss0, ss1)

    def sca(i, b):
        return pltpu.async_copy(ones_v, acc.at[dstl.at[i]], ssem[b],
                                add=True)

    def wait_s(b):
        pltpu.make_async_copy(ones_v, acc.at[dstl.at[0]], ssem[b]).wait()

    sca(0, 0)
    sca(1, 1)

    def pair(o, _):
        wait_s(0)
        sca(2 * o + 2, 0)
        wait_s(1)
        sca(2 * o + 3, 1)
        return 0
    lax.fori_loop(0, ITERS // 2 - 1, pair, 0)
    wait_s(0)
    wait_s(1)

    plsc.subcore_barrier()

    def drain(k, _):
        b = row0 + k * 80
        pltpu.sync_copy(acc.at[pl.ds(b, 80)], deg_out.at[cid, pl.ds(b, 80)])
        return 0
    lax.fori_loop(0, nd, drain, 0)


_MESH = plsc.VectorSubcoreMesh(core_axis_name="c", subcore_axis_name="s",
                               num_cores=NC, num_subcores=NS)

_segsum = pl.kernel(
    _segsum_body,
    out_type=jax.ShapeDtypeStruct((NC, N, D), f32),
    mesh=_MESH,
    scratch_types=(pltpu.VMEM((SG,), jnp.int32),) * (2 * NB)
    + (pltpu.VMEM((SG, D), f32),) * NB
    + (pltpu.VMEM_SHARED((NACC, D), f32),)
    + (pltpu.SemaphoreType.DMA,) * (2 * NB))

_deg_count = pl.kernel(
    _deg_body,
    out_type=jax.ShapeDtypeStruct((NC, N, D), f32),
    mesh=_MESH,
    scratch_types=(pltpu.VMEM((ITERS, G), jnp.int32),
                   pltpu.VMEM((G, D), f32),
                   pltpu.VMEM((16, D), f32),
                   pltpu.VMEM_SHARED((NACC, D), f32),
                   pltpu.SemaphoreType.DMA,
                   pltpu.SemaphoreType.DMA))


def _stage_a_body(tau_ref, x_ref, sxp_ref, degp_ref, h_ref, dinv_ref):
    deg = jnp.maximum(degp_ref[0, :, 0] + degp_ref[1, :, 0], 1.0)
    dinv = (1.0 / deg)[:, None]
    s = sxp_ref[0] + sxp_ref[1]
    xb = x_ref[...]
    h_ref[...] = xb + tau_ref[0, 0] * (s * dinv - xb)
    dinv_ref[...] = jnp.broadcast_to(dinv, (dinv.shape[0], 8))


def _stage_bc_body(h_ref, sp_ref, dinv_ref, w_ref, g_ref):
    dinv = dinv_ref[:, 0][:, None]
    msg = (sp_ref[0] + sp_ref[1]) * dinv - h_ref[...]
    g_ref[...] = jnp.tanh(jnp.dot(msg, w_ref[...],
                                  precision=lax.Precision.HIGHEST))


def _stage_d_body(g1_ref, g2_ref, s1p_ref, s2p_ref, dinv_ref,
                  wconv_ref, wmlp_ref, out_ref):
    dinv = dinv_ref[:, 0][:, None]
    cat = jnp.concatenate(
        [g1_ref[...], g2_ref[...],
         (s1p_ref[0] + s1p_ref[1]) * dinv,
         (s2p_ref[0] + s2p_ref[1]) * dinv], axis=1)
    pre = jax.nn.relu(jnp.dot(cat, wconv_ref[...],
                              precision=lax.Precision.HIGHEST))
    out_ref[...] = jnp.dot(pre, wmlp_ref[...],
                           precision=lax.Precision.HIGHEST)


def _blk():
    return pl.BlockSpec((BN, D), lambda i: (i, 0))


def _pblk():
    return pl.BlockSpec((NC, BN, D), lambda i: (0, i, 0))


def _dinv_blk():
    return pl.BlockSpec((BN, 8), lambda i: (i, 0))


def _full(shape):
    return pl.BlockSpec(shape, lambda i: (0, 0))


_GRID = (N // BN,)


def _stage_a(x, sxp, degp, tau2):
    return pl.pallas_call(
        _stage_a_body,
        grid=_GRID,
        in_specs=[pl.BlockSpec(memory_space=pltpu.SMEM),
                  _blk(), _pblk(), _pblk()],
        out_specs=[_blk(), _dinv_blk()],
        out_shape=[jax.ShapeDtypeStruct((N, D), f32),
                   jax.ShapeDtypeStruct((N, 8), f32)],
    )(tau2, x, sxp, degp)


def _stage_bc(h, sp, dinv, w):
    return pl.pallas_call(
        _stage_bc_body,
        grid=_GRID,
        in_specs=[_blk(), _pblk(), _dinv_blk(), _full((D, D))],
        out_specs=_blk(),
        out_shape=jax.ShapeDtypeStruct((N, D), f32),
    )(h, sp, dinv, w)


def _stage_d(g1, g2, s1p, s2p, dinv, wconv, wmlp):
    return pl.pallas_call(
        _stage_d_body,
        grid=_GRID,
        in_specs=[_blk(), _blk(), _pblk(), _pblk(), _dinv_blk(),
                  _full((4 * D, D)), _full((D, OUT))],
        out_specs=pl.BlockSpec((BN, OUT), lambda i: (i, 0)),
        out_shape=jax.ShapeDtypeStruct((N, OUT), f32),
    )(g1, g2, s1p, s2p, dinv, wconv, wmlp)


def kernel(x, edge_index, tau, Wg0, Wg1, Wconv, Wmlp):
    src = edge_index[0].astype(jnp.int32)
    dst = edge_index[1].astype(jnp.int32)
    nw = NC * NS
    npad = SPER_SUB - E // nw
    psrc = jnp.zeros((nw, npad), jnp.int32)
    pdst = (N + 8 * jnp.arange(nw, dtype=jnp.int32)[:, None]
            + jnp.arange(npad, dtype=jnp.int32)[None, :] % 8)
    src = jnp.concatenate([src.reshape(nw, -1), psrc], axis=1).reshape(-1)
    dst = jnp.concatenate([dst.reshape(nw, -1), pdst], axis=1).reshape(-1)
    dst2 = dst.reshape(nw, ITERS, G)
    tau2 = jnp.reshape(tau, (1, 1)).astype(f32)

    degp = _deg_count(dst2)
    sxp = _segsum(x, src, dst)
    h, dinv = _stage_a(x, sxp, degp, tau2)
    shp = _segsum(h, src, dst)
    g1 = _stage_bc(h, shp, dinv, Wg0)
    sg1p = _segsum(g1, src, dst)
    g2 = _stage_bc(g1, sg1p, dinv, Wg1)
    sg2p = _segsum(g2, src, dst)
    return _stage_d(g1, g2, sg1p, sg2p, dinv, Wconv, Wmlp)


# speedup vs baseline: 3.0257x; 2.5080x over previous
import jax
import jax.numpy as jnp
from jax import lax
from jax.experimental import pallas as pl
from jax.experimental.pallas import tpu as pltpu
from jax.experimental.pallas import tpu_sc as plsc

N = 10000
E = 320000
D = 128
OUT = 64

NC = 2
NS = 16
G = 128
ITERS = 80
EPAD = NC * NS * ITERS * G
SG = 80
SITERS = 128
SPER_SUB = SG * SITERS
NACC = N + 256
ROWS_SUB = 640
ROWS_LAST = N - ROWS_SUB * (NS - 1)

BN = 1000
f32 = jnp.float32


def _zero_fill(zbuf):
    z16 = jnp.zeros((16,), f32)
    for r in range(16):
        for c in range(D // 16):
            zbuf[r, pl.ds(c * 16, 16)] = z16


def _acc_chunks(sid):
    nz = jnp.where(sid < NS - 1, ROWS_SUB // 16, ROWS_LAST // 16)
    nd = jnp.where(sid < NS - 1, ROWS_SUB // 80, ROWS_LAST // 80)
    return sid * ROWS_SUB, nz, nd


NB = 4


def _segsum_body(table, src, dst, s_out,
                 srcg0, srcg1, srcg2, srcg3,
                 dstg0, dstg1, dstg2, dstg3,
                 rows0, rows1, rows2, rows3, acc,
                 gs0, gs1, gs2, gs3, ss0, ss1, ss2, ss3):
    cid = lax.axis_index("c")
    sid = lax.axis_index("s")
    wid = cid * NS + sid
    row0, nz, nd = _acc_chunks(sid)
    base0 = wid * SPER_SUB

    srcg = (srcg0, srcg1, srcg2, srcg3)
    dstg = (dstg0, dstg1, dstg2, dstg3)
    rows = (rows0, rows1, rows2, rows3)
    gsem = (gs0, gs1, gs2, gs3)
    ssem = (ss0, ss1, ss2, ss3)

    def ld_src(i, p):
        pltpu.sync_copy(src.at[pl.ds(base0 + i * SG, SG)], srcg[p])

    def ld_dst(i, p):
        pltpu.sync_copy(dst.at[pl.ds(base0 + i * SG, SG)], dstg[p])

    def gat(b):
        pltpu.async_copy(table.at[srcg[b]], rows[b], gsem[b])

    def sca(b):
        pltpu.async_copy(rows[b], acc.at[dstg[b]], ssem[b], add=True)

    def wait_g(b):
        pltpu.make_async_copy(table.at[srcg[b]], rows[b], gsem[b]).wait()

    def wait_s(b):
        pltpu.make_async_copy(rows[b], acc.at[dstg[b]], ssem[b]).wait()

    _zero_fill(rows0)
    lax.fori_loop(
        0, nz,
        lambda k, _: (pltpu.sync_copy(rows0.at[pl.ds(0, 16)],
                                      acc.at[pl.ds(row0 + k * 16, 16)]),
                      0)[1], 0)
    plsc.subcore_barrier()

    for i in range(NB - 1):
        ld_src(i, i)
        gat(i)

    def quad(o, _):
        for j in range(NB):
            i = NB * o + j
            nb = (j + NB - 1) % NB
            wait_g(j)
            ld_dst(i, j)
            sca(j)
            if j == 0:
                @pl.when(o >= 1)
                def _():
                    wait_s(nb)
                ld_src(i + NB - 1, nb)
                gat(nb)
            else:
                @pl.when(i + NB - 1 < SITERS)
                def _():
                    wait_s(nb)
                    ld_src(i + NB - 1, nb)
                    gat(nb)
        return 0
    lax.fori_loop(0, SITERS // NB, quad, 0)

    for b in range(NB):
        wait_s(b)

    plsc.subcore_barrier()

    def drain(k, _):
        b = row0 + k * 80
        pltpu.sync_copy(acc.at[pl.ds(b, 80)], s_out.at[cid, pl.ds(b, 80)])
        return 0
    lax.fori_loop(0, nd, drain, 0)


def _deg_body(dst2, deg_out, dstl, ones_v, zbuf, acc, ss0, ss1):
    cid = lax.axis_index("c")
    sid = lax.axis_index("s")
    wid = cid * NS + sid
    row0, nz, nd = _acc_chunks(sid)

    pltpu.sync_copy(dst2.at[wid], dstl)

    _zero_fill(zbuf)
    one16 = jnp.ones((16,), f32)
    for r in range(G):
        for c in range(D // 16):
            ones_v[r, pl.ds(c * 16, 16)] = one16
    lax.fori_loop(
        0, nz,
        lambda k, _: (pltpu.sync_copy(zbuf, acc.at[pl.ds(row0 + k * 16, 16)]),
                      0)[1], 0)
    plsc.subcore_barrier()

    ssem = (ss0, ss1)

    def sca(i, b):
        return pltpu.async_copy(ones_v, acc.at[dstl.at[i]], ssem[b],
                                add=True)

    def wait_s(b):
        pltpu.make_async_copy(ones_v, acc.at[dstl.at[0]], ssem[b]).wait()

    sca(0, 0)
    sca(1, 1)

    def pair(o, _):
        wait_s(0)
        sca(2 * o + 2, 0)
        wait_s(1)
        sca(2 * o + 3, 1)
        return 0
    lax.fori_loop(0, ITERS // 2 - 1, pair, 0)
    wait_s(0)
    wait_s(1)

    plsc.subcore_barrier()

    def drain(k, _):
        b = row0 + k * 80
        pltpu.sync_copy(acc.at[pl.ds(b, 80)], deg_out.at[cid, pl.ds(b, 80)])
        return 0
    lax.fori_loop(0, nd, drain, 0)


_MESH = plsc.VectorSubcoreMesh(core_axis_name="c", subcore_axis_name="s",
                               num_cores=NC, num_subcores=NS)

_segsum = pl.kernel(
    _segsum_body,
    out_type=jax.ShapeDtypeStruct((NC, N, D), f32),
    mesh=_MESH,
    scratch_types=(pltpu.VMEM((SG,), jnp.int32),) * (2 * NB)
    + (pltpu.VMEM((SG, D), f32),) * NB
    + (pltpu.VMEM_SHARED((NACC, D), f32),)
    + (pltpu.SemaphoreType.DMA,) * (2 * NB))

_deg_count = pl.kernel(
    _deg_body,
    out_type=jax.ShapeDtypeStruct((NC, N, D), f32),
    mesh=_MESH,
    scratch_types=(pltpu.VMEM((ITERS, G), jnp.int32),
                   pltpu.VMEM((G, D), f32),
                   pltpu.VMEM((16, D), f32),
                   pltpu.VMEM_SHARED((NACC, D), f32),
                   pltpu.SemaphoreType.DMA,
                   pltpu.SemaphoreType.DMA))


def _stage_a_body(tau_ref, x_ref, sxp_ref, degp_ref, h_ref, dinv_ref):
    deg = jnp.maximum(degp_ref[0, :, 0] + degp_ref[1, :, 0], 1.0)
    dinv = (1.0 / deg)[:, None]
    s = sxp_ref[0] + sxp_ref[1]
    xb = x_ref[...]
    h_ref[...] = xb + tau_ref[0, 0] * (s * dinv - xb)
    dinv_ref[...] = jnp.broadcast_to(dinv, (dinv.shape[0], 8))


def _stage_bc_body(h_ref, sp_ref, dinv_ref, w_ref, g_ref):
    dinv = dinv_ref[:, 0][:, None]
    msg = (sp_ref[0] + sp_ref[1]) * dinv - h_ref[...]
    g_ref[...] = jnp.tanh(jnp.dot(msg, w_ref[...],
                                  precision=lax.Precision.HIGHEST))


def _stage_d_body(g1_ref, g2_ref, s1p_ref, s2p_ref, dinv_ref,
                  wconv_ref, wmlp_ref, out_ref):
    dinv = dinv_ref[:, 0][:, None]
    cat = jnp.concatenate(
        [g1_ref[...], g2_ref[...],
         (s1p_ref[0] + s1p_ref[1]) * dinv,
         (s2p_ref[0] + s2p_ref[1]) * dinv], axis=1)
    pre = jax.nn.relu(jnp.dot(cat, wconv_ref[...],
                              precision=lax.Precision.HIGHEST))
    out_ref[...] = jnp.dot(pre, wmlp_ref[...],
                           precision=lax.Precision.HIGHEST)


def _blk():
    return pl.BlockSpec((BN, D), lambda i: (i, 0))


def _pblk():
    return pl.BlockSpec((NC, BN, D), lambda i: (0, i, 0))


def _dinv_blk():
    return pl.BlockSpec((BN, 8), lambda i: (i, 0))


def _full(shape):
    return pl.BlockSpec(shape, lambda i: (0, 0))


_GRID = (N // BN,)


def _stage_a(x, sxp, degp, tau2):
    return pl.pallas_call(
        _stage_a_body,
        grid=_GRID,
        in_specs=[pl.BlockSpec(memory_space=pltpu.SMEM),
                  _blk(), _pblk(), _pblk()],
        out_specs=[_blk(), _dinv_blk()],
        out_shape=[jax.ShapeDtypeStruct((N, D), f32),
                   jax.ShapeDtypeStruct((N, 8), f32)],
    )(tau2, x, sxp, degp)


def _stage_bc(h, sp, dinv, w):
    return pl.pallas_call(
        _stage_bc_body,
        grid=_GRID,
        in_specs=[_blk(), _pblk(), _dinv_blk(), _full((D, D))],
        out_specs=_blk(),
        out_shape=jax.ShapeDtypeStruct((N, D), f32),
    )(h, sp, dinv, w)


def _stage_d(g1, g2, s1p, s2p, dinv, wconv, wmlp):
    return pl.pallas_call(
        _stage_d_body,
        grid=_GRID,
        in_specs=[_blk(), _blk(), _pblk(), _pblk(), _dinv_blk(),
                  _full((4 * D, D)), _full((D, OUT))],
        out_specs=pl.BlockSpec((BN, OUT), lambda i: (i, 0)),
        out_shape=jax.ShapeDtypeStruct((N, OUT), f32),
    )(g1, g2, s1p, s2p, dinv, wconv, wmlp)


def kernel(x, edge_index, tau, Wg0, Wg1, Wconv, Wmlp):
    src = edge_index[0].astype(jnp.int32)
    dst = edge_index[1].astype(jnp.int32)
    nw = NC * NS
    npad = SPER_SUB - E // nw
    psrc = (8 * jnp.arange(nw, dtype=jnp.int32)[:, None]
            + jnp.arange(npad, dtype=jnp.int32)[None, :] % 8)
    pdst = (N + 8 * jnp.arange(nw, dtype=jnp.int32)[:, None]
            + jnp.arange(npad, dtype=jnp.int32)[None, :] % 8)
    src = jnp.concatenate([src.reshape(nw, -1), psrc], axis=1).reshape(-1)
    dst = jnp.concatenate([dst.reshape(nw, -1), pdst], axis=1).reshape(-1)
    dst2 = dst.reshape(nw, ITERS, G)
    tau2 = jnp.reshape(tau, (1, 1)).astype(f32)

    degp = _deg_count(dst2)
    sxp = _segsum(x, src, dst)
    h, dinv = _stage_a(x, sxp, degp, tau2)
    shp = _segsum(h, src, dst)
    g1 = _stage_bc(h, shp, dinv, Wg0)
    sg1p = _segsum(g1, src, dst)
    g2 = _stage_bc(g1, sg1p, dinv, Wg1)
    sg2p = _segsum(g2, src, dst)
    return _stage_d(g1, g2, sg1p, sg2p, dinv, Wconv, Wmlp)
